# baseline (speedup 1.0000x reference)
import jax
import jax.numpy as jnp
from jax.experimental import pallas as pl
from jax.experimental.pallas import tpu as pltpu

BM = 640


def _s0_kernel(h_ref, w0_ref, out_ref):
    out_ref[...] = jnp.dot(
        h_ref[...].astype(jnp.bfloat16),
        w0_ref[...].astype(jnp.bfloat16),
        preferred_element_type=jnp.float32,
    ).astype(jnp.bfloat16)


def _layer1_kernel(a_ref, s0_ref, b0_ref, w12_ref, out_ref):
    acc = jnp.dot(
        a_ref[...].astype(jnp.bfloat16),
        s0_ref[...],
        preferred_element_type=jnp.float32,
    )
    hh = jnp.maximum(acc + b0_ref[...], 0.0)
    out_ref[...] = jnp.dot(
        hh.astype(jnp.bfloat16),
        w12_ref[...],
        preferred_element_type=jnp.float32,
    ).astype(jnp.bfloat16)


def _layer2_kernel(a_ref, s12_ref, b12_ref, out_ref):
    acc = jnp.dot(
        a_ref[...].astype(jnp.bfloat16),
        s12_ref[...],
        preferred_element_type=jnp.float32,
    )
    out_ref[...] = acc + b12_ref[...]


@jax.jit
def kernel(H, A, W0, b0, W1, b1, W2, b2):
    n, nfeat = H.shape
    nhid = W0.shape[1]
    nclass = W1.shape[1]
    ndim = W2.shape[1]
    nm = -(-n // BM)

    s0 = pl.pallas_call(
        _s0_kernel,
        grid=(nm,),
        in_specs=[
            pl.BlockSpec((BM, nfeat), lambda i: (i, 0)),
            pl.BlockSpec((nfeat, nhid), lambda i: (0, 0)),
        ],
        out_specs=pl.BlockSpec((BM, nhid), lambda i: (i, 0)),
        out_shape=jax.ShapeDtypeStruct((n, nhid), jnp.bfloat16),
    )(H, W0)

    w12 = jnp.concatenate([W1, W2], axis=1).astype(jnp.bfloat16)
    b12 = jnp.concatenate([b1, b2])[None, :]
    b0_2d = b0[None, :]
    ncat = nclass + ndim

    a_spec = pl.BlockSpec((BM, n), lambda i: (i, 0))
    full_spec = lambda shape: pl.BlockSpec(shape, lambda i: (0, 0))
    out_spec = lambda width: pl.BlockSpec((BM, width), lambda i: (i, 0))
    cparams = pltpu.CompilerParams(dimension_semantics=("arbitrary",))

    s12 = pl.pallas_call(
        _layer1_kernel,
        grid=(nm,),
        in_specs=[
            a_spec,
            full_spec((n, nhid)),
            full_spec((1, nhid)),
            full_spec((nhid, ncat)),
        ],
        out_specs=out_spec(ncat),
        out_shape=jax.ShapeDtypeStruct((n, ncat), jnp.bfloat16),
        compiler_params=cparams,
    )(A, s0, b0_2d, w12)

    out = pl.pallas_call(
        _layer2_kernel,
        grid=(nm,),
        in_specs=[
            a_spec,
            full_spec((n, ncat)),
            full_spec((1, ncat)),
        ],
        out_specs=out_spec(ncat),
        out_shape=jax.ShapeDtypeStruct((n, ncat), jnp.float32),
        compiler_params=cparams,
    )(A, s12, b12)

    return (out[:, :nclass], out[:, nclass:])


# speedup vs baseline: 1.0275x; 1.0275x over previous
import functools

import jax
import jax.numpy as jnp
from jax.experimental import pallas as pl
from jax.experimental.pallas import tpu as pltpu

BM = 400


def _s0_kernel(h_ref, w0_ref, out_ref):
    out_ref[...] = jnp.dot(
        h_ref[...].astype(jnp.bfloat16),
        w0_ref[...].astype(jnp.bfloat16),
        preferred_element_type=jnp.float32,
    ).astype(jnp.bfloat16)


def _fused_kernel(a_ref, s0_ref, b0_ref, w12_ref, b12_ref, out_ref, s12_ref,
                  *, nm):
    t = pl.program_id(0)

    @pl.when(t < nm)
    def _phase1():
        acc = jnp.dot(
            a_ref[...].astype(jnp.bfloat16),
            s0_ref[...],
            preferred_element_type=jnp.float32,
        )
        hh = jnp.maximum(acc + b0_ref[...], 0.0).astype(jnp.bfloat16)
        s12_ref[pl.ds(t * BM, BM), :] = jnp.dot(
            hh,
            w12_ref[...],
            preferred_element_type=jnp.float32,
        ).astype(jnp.bfloat16)

    @pl.when(t >= nm)
    def _phase2():
        acc = jnp.dot(
            a_ref[...].astype(jnp.bfloat16),
            s12_ref[...],
            preferred_element_type=jnp.float32,
        )
        out_ref[...] = acc + b12_ref[...]


@jax.jit
def kernel(H, A, W0, b0, W1, b1, W2, b2):
    n, nfeat = H.shape
    nhid = W0.shape[1]
    nclass = W1.shape[1]
    ndim = W2.shape[1]
    nm = n // BM

    s0 = pl.pallas_call(
        _s0_kernel,
        grid=(nm,),
        in_specs=[
            pl.BlockSpec((BM, nfeat), lambda i: (i, 0)),
            pl.BlockSpec((nfeat, nhid), lambda i: (0, 0)),
        ],
        out_specs=pl.BlockSpec((BM, nhid), lambda i: (i, 0)),
        out_shape=jax.ShapeDtypeStruct((n, nhid), jnp.bfloat16),
    )(H, W0)

    w12 = jnp.concatenate([W1, W2], axis=1).astype(jnp.bfloat16)
    b12 = jnp.concatenate([b1, b2])[None, :]
    b0_2d = b0[None, :]
    ncat = nclass + ndim

    full_spec = lambda shape: pl.BlockSpec(shape, lambda t: (0, 0))

    out = pl.pallas_call(
        functools.partial(_fused_kernel, nm=nm),
        grid=(2 * nm,),
        in_specs=[
            pl.BlockSpec((BM, n), lambda t: (jnp.where(t < nm, t, t - nm), 0)),
            full_spec((n, nhid)),
            full_spec((1, nhid)),
            full_spec((nhid, ncat)),
            full_spec((1, ncat)),
        ],
        out_specs=pl.BlockSpec((BM, ncat), lambda t: (jnp.maximum(t - nm, 0), 0)),
        out_shape=jax.ShapeDtypeStruct((n, ncat), jnp.float32),
        scratch_shapes=[pltpu.VMEM((n, ncat), jnp.bfloat16)],
        compiler_params=pltpu.CompilerParams(dimension_semantics=("arbitrary",)),
    )(A, s0, b0_2d, w12, b12)

    return (out[:, :nclass], out[:, nclass:])
